# baseline (speedup 1.0000x reference)
import jax
import jax.numpy as jnp
from jax import lax
from jax.experimental import pallas as pl
from jax.experimental.pallas import tpu as pltpu
from jax.experimental.pallas import tpu_sc as plsc

B = 4
C = 64
P = 12000
H = 512
W = 512

NC = 2
NS = 16
L = 16

CHUNKS_PER_BATCH = (NC * NS) // B
ROWS_PER_CHUNK = H // CHUNKS_PER_BATCH
CHUNK_PIX = ROWS_PER_CHUNK * W

PGROUPS = P // L
MGROUPS = CHUNK_PIX // L

STG0 = 0
STG1 = CHUNK_PIX + L
F_OFF = 0
XI_OFF = P
YI_OFF = 2 * P
MAP_OFF = 3 * P
PL_OFF = MAP_OFF + CHUNK_PIX
LI_OFF = PL_OFF + P + L
SH_OFF = LI_OFF + P + L
XC0 = SH_OFF + 24
XC1 = XC0 + P
POOL = XC1 + P

INT_MAX = 2**31 - 1


def _i(v):
  return plsc.bitcast(v, jnp.int32)


def _f(v):
  return plsc.bitcast(v, jnp.float32)


def _body(x_hbm, ind_hbm, out_hbm, pool, sx0, sx1, ss0, ss1):
  cid = lax.axis_index("c")
  sid = lax.axis_index("s")
  wid = cid * NS + sid
  b = wid // CHUNKS_PER_BATCH
  chunk = wid % CHUNKS_PER_BATCH
  lo = chunk * CHUNK_PIX
  iota = lax.iota(jnp.int32, L)
  fzeros = jnp.zeros((L,), jnp.float32)

  pltpu.async_copy(x_hbm.at[b, 0], pool.at[pl.ds(XC0, P)], sx0)
  pltpu.async_copy(x_hbm.at[b, 1], pool.at[pl.ds(XC1, P)], sx1)
  pltpu.async_copy(ind_hbm.at[0, b], pool.at[pl.ds(F_OFF, P)], ss0)
  pltpu.async_copy(ind_hbm.at[1, b], pool.at[pl.ds(XI_OFF, P)], ss0)
  pltpu.async_copy(ind_hbm.at[2, b], pool.at[pl.ds(YI_OFF, P)], ss0)

  def zero_map(i, _):
    pool[pl.ds(MAP_OFF + i * L, L)] = fzeros
    return _
  lax.fori_loop(0, MGROUPS, zero_map, None)

  for off in (F_OFF, XI_OFF, YI_OFF):
    pltpu.make_async_copy(
        ind_hbm.at[0, b], pool.at[pl.ds(off, P)], ss0).wait()

  def point_group(g, _):
    base = g * L
    f = _i(pool[pl.ds(F_OFF + base, L)])
    xv = _i(pool[pl.ds(XI_OFF + base, L)])
    yv = _i(pool[pl.ds(YI_OFF + base, L)])
    local = yv * W + xv - lo
    valid = (f != 0) & (local >= 0) & (local < CHUNK_PIX)
    key = jnp.where(valid, local * L + iota, INT_MAX)
    skey, sp = plsc.sort_key_val(key, base + iota)
    spix = lax.shift_right_arithmetic(skey, 4)
    pool[pl.ds(SH_OFF, L)] = _f(jnp.full((L,), -1, jnp.int32))
    plsc.store_scatter(
        pool, [SH_OFF + jnp.maximum(iota - 1, 0)], _f(spix), mask=iota > 0)
    npix = _i(pool[pl.ds(SH_OFF, L)])
    winner = (spix != npix) & (skey != INT_MAX)
    plsc.store_scatter(
        pool, [MAP_OFF + (spix & (CHUNK_PIX - 1))], _f(sp + 1), mask=winner)
    return _
  lax.fori_loop(0, PGROUPS, point_group, None)

  def compact(mg, cnt):
    m = _i(pool[pl.ds(MAP_OFF + mg * L, L)])
    msk = m > 0
    mi = msk.astype(jnp.int32)
    pos = jnp.maximum(cnt + plsc.cumsum(mi) - 1, 0)
    plsc.store_scatter(pool, [PL_OFF + pos], _f(m - 1), mask=msk)
    plsc.store_scatter(pool, [LI_OFF + pos], _f(mg * L + iota), mask=msk)
    return cnt + jnp.sum(mi)
  cnt = lax.fori_loop(0, MGROUPS, compact, jnp.int32(0))

  pool[pl.ds(PL_OFF + cnt, L)] = fzeros
  pool[pl.ds(LI_OFF + cnt, L)] = _f(CHUNK_PIX + iota)
  ngroups = (cnt + (L - 1)) // L

  def zero_stage(i, _):
    pool[pl.ds(i * L, L)] = fzeros
    return _
  lax.fori_loop(0, 2 * (MGROUPS + 1), zero_stage, None)

  r0 = chunk * ROWS_PER_CHUNK

  def stg_src(stg):
    return pool.at[pl.ds(stg, CHUNK_PIX)]

  def out_dst(c):
    return out_hbm.at[b, c, pl.ds(lo, CHUNK_PIX)]

  def do_channel(c, xc, stg, sem_x, sem_s):
    pltpu.make_async_copy(
        x_hbm.at[b, c], pool.at[pl.ds(xc, P)], sem_x).wait()

    @pl.when(c >= 2)
    def _wait_stage():
      pltpu.make_async_copy(stg_src(stg), out_dst(c - 2), sem_s).wait()

    def gather_group(g, _):
      pv = _i(pool[pl.ds(PL_OFF + g * L, L)])
      li = _i(pool[pl.ds(LI_OFF + g * L, L)])
      vals = plsc.load_gather(pool, [xc + pv])
      plsc.store_scatter(pool, [stg + li], vals)
      return _
    lax.fori_loop(0, ngroups, gather_group, None)

    @pl.when(c < C - 2)
    def _prefetch():
      pltpu.async_copy(x_hbm.at[b, c + 2], pool.at[pl.ds(xc, P)], sem_x)

    pltpu.async_copy(stg_src(stg), out_dst(c), sem_s)

  def chan_pair(cc, _):
    do_channel(cc * 2, XC0, STG0, sx0, ss0)
    do_channel(cc * 2 + 1, XC1, STG1, sx1, ss1)
    return _
  lax.fori_loop(0, C // 2, chan_pair, None)

  pltpu.make_async_copy(stg_src(STG0), out_dst(C - 2), ss0).wait()
  pltpu.make_async_copy(stg_src(STG1), out_dst(C - 1), ss1).wait()


_mesh = plsc.VectorSubcoreMesh(
    core_axis_name="c", subcore_axis_name="s", num_cores=NC, num_subcores=NS)

_sc_scatter = pl.kernel(
    _body,
    out_type=jax.ShapeDtypeStruct((B, C, H * W), jnp.float32),
    mesh=_mesh,
    scratch_types=[
        pltpu.VMEM((POOL,), jnp.float32),
        pltpu.SemaphoreType.DMA,
        pltpu.SemaphoreType.DMA,
        pltpu.SemaphoreType.DMA,
        pltpu.SemaphoreType.DMA,
    ],
    compiler_params=pltpu.CompilerParams(
        needs_layout_passes=False, use_tc_tiling_on_sc=False),
)


@jax.jit
def kernel(x, inds):
  ind_t = lax.bitcast_convert_type(
      jnp.transpose(inds, (2, 0, 1)), jnp.float32)
  return _sc_scatter(x, ind_t).reshape(B, C, H, W)


# speedup vs baseline: 1.0056x; 1.0056x over previous
import jax
import jax.numpy as jnp
from jax import lax
from jax.experimental import pallas as pl
from jax.experimental.pallas import tpu as pltpu
from jax.experimental.pallas import tpu_sc as plsc

B = 4
C = 64
P = 12000
H = 512
W = 512

NC = 2
NS = 16
L = 16

CHUNKS_PER_BATCH = (NC * NS) // B
ROWS_PER_CHUNK = H // CHUNKS_PER_BATCH
CHUNK_PIX = ROWS_PER_CHUNK * W

PGROUPS = P // L
MGROUPS = CHUNK_PIX // L

F_OFF = 0
XI_OFF = P
YI_OFF = 2 * P
XC0 = 0
XC1 = P
POOL = 3 * P

MAP_ROW = 0
STG0_ROW = 0
STG1_ROW = 64
LIST_ROW = 128
SH_ROW = 152
GRID_ROWS = 153

INT_MAX = 2**31 - 1
PBITS = 14


def _i(v):
  return plsc.bitcast(v, jnp.int32)


def _f(v):
  return plsc.bitcast(v, jnp.float32)


def _body(x_hbm, ind_hbm, out_hbm, pool, grid, sx0, sx1, ss0, ss1):
  cid = lax.axis_index("c")
  sid = lax.axis_index("s")
  wid = cid * NS + sid
  b = wid // CHUNKS_PER_BATCH
  chunk = wid % CHUNKS_PER_BATCH
  lo = chunk * CHUNK_PIX
  r0 = chunk * ROWS_PER_CHUNK
  iota = lax.iota(jnp.int32, L)
  fzeros = jnp.zeros((L,), jnp.float32)
  row_sh = jnp.full((L,), SH_ROW, jnp.int32)

  pltpu.async_copy(ind_hbm.at[0, b], pool.at[pl.ds(F_OFF, P)], ss0)
  pltpu.async_copy(ind_hbm.at[1, b], pool.at[pl.ds(XI_OFF, P)], ss0)
  pltpu.async_copy(ind_hbm.at[2, b], pool.at[pl.ds(YI_OFF, P)], ss0)

  def zero_map(i, _):
    grid[i >> 5, pl.ds((i & 31) * L, L)] = fzeros
    return _
  lax.fori_loop(0, MGROUPS, zero_map, None)

  for off in (F_OFF, XI_OFF, YI_OFF):
    pltpu.make_async_copy(
        ind_hbm.at[0, b], pool.at[pl.ds(off, P)], ss0).wait()

  def point_group(g, _):
    base = g * L
    f = _i(pool[pl.ds(F_OFF + base, L)])
    xv = _i(pool[pl.ds(XI_OFF + base, L)])
    yv = _i(pool[pl.ds(YI_OFF + base, L)])
    local = yv * W + xv - lo
    valid = (f != 0) & (local >= 0) & (local < CHUNK_PIX)
    key = jnp.where(valid, local * L + iota, INT_MAX)
    skey, sp = plsc.sort_key_val(key, base + iota)
    spix = lax.shift_right_arithmetic(skey, 4)
    grid[SH_ROW, pl.ds(0, L)] = _f(jnp.full((L,), -1, jnp.int32))
    plsc.store_scatter(
        grid, [row_sh, jnp.maximum(iota - 1, 0)], _f(spix), mask=iota > 0)
    npix = _i(grid[SH_ROW, pl.ds(0, L)])
    winner = (spix != npix) & (skey != INT_MAX)
    spix_c = spix & (CHUNK_PIX - 1)
    plsc.store_scatter(
        grid, [spix_c >> 9, spix_c & (W - 1)], _f(sp + 1), mask=winner)
    return _
  lax.fori_loop(0, PGROUPS, point_group, None)

  pltpu.async_copy(x_hbm.at[b, 0], pool.at[pl.ds(XC0, P)], sx0)
  pltpu.async_copy(x_hbm.at[b, 1], pool.at[pl.ds(XC1, P)], sx1)

  def compact(mg, cnt):
    m = _i(grid[mg >> 5, pl.ds((mg & 31) * L, L)])
    msk = m > 0
    mi = msk.astype(jnp.int32)
    pos = jnp.maximum(cnt + plsc.cumsum(mi) - 1, 0)
    entry = ((mg * L + iota) << PBITS) + m - 1
    plsc.store_scatter(
        grid, [LIST_ROW + (pos >> 9), pos & (W - 1)], _f(entry), mask=msk)
    return cnt + jnp.sum(mi)
  cnt = lax.fori_loop(0, MGROUPS, compact, jnp.int32(0))

  pad = cnt + iota
  plsc.store_scatter(
      grid, [LIST_ROW + (pad >> 9), pad & (W - 1)],
      _f(jnp.full((L,), -1, jnp.int32)))
  ngroups = (cnt + (L - 1)) // L

  def zero_stage(i, _):
    grid[i >> 5, pl.ds((i & 31) * L, L)] = fzeros
    return _
  lax.fori_loop(0, 2 * MGROUPS, zero_stage, None)

  def out_dst(c):
    return out_hbm.at[b, c, pl.ds(r0, ROWS_PER_CHUNK)]

  def do_channel(c, xc, srow, sem_x, sem_s):
    pltpu.make_async_copy(
        x_hbm.at[b, c], pool.at[pl.ds(xc, P)], sem_x).wait()

    @pl.when(c >= 2)
    def _wait_stage():
      pltpu.make_async_copy(
          grid.at[pl.ds(srow, ROWS_PER_CHUNK)], out_dst(c - 2), sem_s).wait()

    def gather_group(g, _):
      e = _i(grid[LIST_ROW + (g >> 5), pl.ds((g & 31) * L, L)])
      live = e >= 0
      pv = e & ((1 << PBITS) - 1)
      li = (e >> PBITS) & (CHUNK_PIX - 1)
      vals = plsc.load_gather(pool, [xc + pv], mask=live)
      plsc.store_scatter(
          grid, [srow + (li >> 9), li & (W - 1)], vals, mask=live)
      return _
    lax.fori_loop(0, ngroups, gather_group, None)

    @pl.when(c < C - 2)
    def _prefetch():
      pltpu.async_copy(x_hbm.at[b, c + 2], pool.at[pl.ds(xc, P)], sem_x)

    pltpu.async_copy(grid.at[pl.ds(srow, ROWS_PER_CHUNK)], out_dst(c), sem_s)

  def chan_pair(cc, _):
    do_channel(cc * 2, XC0, STG0_ROW, sx0, ss0)
    do_channel(cc * 2 + 1, XC1, STG1_ROW, sx1, ss1)
    return _
  lax.fori_loop(0, C // 2, chan_pair, None)

  pltpu.make_async_copy(
      grid.at[pl.ds(STG0_ROW, ROWS_PER_CHUNK)], out_dst(C - 2), ss0).wait()
  pltpu.make_async_copy(
      grid.at[pl.ds(STG1_ROW, ROWS_PER_CHUNK)], out_dst(C - 1), ss1).wait()


_mesh = plsc.VectorSubcoreMesh(
    core_axis_name="c", subcore_axis_name="s", num_cores=NC, num_subcores=NS)

_sc_scatter = pl.kernel(
    _body,
    out_type=jax.ShapeDtypeStruct((B, C, H, W), jnp.float32),
    mesh=_mesh,
    scratch_types=[
        pltpu.VMEM((POOL,), jnp.float32),
        pltpu.VMEM((GRID_ROWS, W), jnp.float32),
        pltpu.SemaphoreType.DMA,
        pltpu.SemaphoreType.DMA,
        pltpu.SemaphoreType.DMA,
        pltpu.SemaphoreType.DMA,
    ],
    compiler_params=pltpu.CompilerParams(
        needs_layout_passes=False, use_tc_tiling_on_sc=False),
)


@jax.jit
def kernel(x, inds):
  ind_t = lax.bitcast_convert_type(
      jnp.transpose(inds, (2, 0, 1)), jnp.float32)
  return _sc_scatter(x, ind_t)


# speedup vs baseline: 2.0681x; 2.0565x over previous
import jax
import jax.numpy as jnp
from jax import lax
from jax.experimental import pallas as pl
from jax.experimental.pallas import tpu as pltpu
from jax.experimental.pallas import tpu_sc as plsc

B = 4
C = 64
P = 12000
H = 512
W = 512

NC = 2
NS = 16
L = 16

CHUNKS_PER_BATCH = (NC * NS) // B
ROWS_PER_CHUNK = H // CHUNKS_PER_BATCH
CHUNK_PIX = ROWS_PER_CHUNK * W

PGROUPS = P // L
MGROUPS = CHUNK_PIX // L

F_OFF = 0
XI_OFF = P
YI_OFF = 2 * P
XC0 = 0
XC1 = P
POOL = 3 * P

MAP_ROW = 0
STG0_ROW = 0
STG1_ROW = 64
LIST_ROW = 128
SH_ROW = 152
GRID_ROWS = 153

INT_MAX = 2**31 - 1
PBITS = 14


def _i(v):
  return plsc.bitcast(v, jnp.int32)


def _f(v):
  return plsc.bitcast(v, jnp.float32)


def _body(x_hbm, ind_hbm, out_hbm, pool, grid, sx0, sx1, ss0, ss1):
  cid = lax.axis_index("c")
  sid = lax.axis_index("s")
  wid = cid * NS + sid
  b = wid // CHUNKS_PER_BATCH
  chunk = wid % CHUNKS_PER_BATCH
  lo = chunk * CHUNK_PIX
  r0 = chunk * ROWS_PER_CHUNK
  iota = lax.iota(jnp.int32, L)
  fzeros = jnp.zeros((L,), jnp.float32)
  row_sh = jnp.full((L,), SH_ROW, jnp.int32)

  pltpu.async_copy(ind_hbm.at[pl.ds(b * P, P)], pool.at[pl.ds(F_OFF, P)], ss0)
  pltpu.async_copy(
      ind_hbm.at[pl.ds((B + b) * P, P)], pool.at[pl.ds(XI_OFF, P)], ss0)
  pltpu.async_copy(
      ind_hbm.at[pl.ds((2 * B + b) * P, P)], pool.at[pl.ds(YI_OFF, P)], ss0)

  def zero_map(i, _):
    grid[i >> 5, pl.ds((i & 31) * L, L)] = fzeros
    return _
  lax.fori_loop(0, MGROUPS, zero_map, None)

  for off in (F_OFF, XI_OFF, YI_OFF):
    pltpu.make_async_copy(
        ind_hbm.at[pl.ds(b * P, P)], pool.at[pl.ds(off, P)], ss0).wait()

  def point_group(g, _):
    base = g * L
    f = _i(pool[pl.ds(F_OFF + base, L)])
    xv = _i(pool[pl.ds(XI_OFF + base, L)])
    yv = _i(pool[pl.ds(YI_OFF + base, L)])
    local = yv * W + xv - lo
    valid = (f != 0) & (local >= 0) & (local < CHUNK_PIX)
    key = jnp.where(valid, local * L + iota, INT_MAX)
    skey, sp = plsc.sort_key_val(key, base + iota)
    spix = lax.shift_right_arithmetic(skey, 4)
    grid[SH_ROW, pl.ds(0, L)] = _f(jnp.full((L,), -1, jnp.int32))
    plsc.store_scatter(
        grid, [row_sh, jnp.maximum(iota - 1, 0)], _f(spix), mask=iota > 0)
    npix = _i(grid[SH_ROW, pl.ds(0, L)])
    winner = (spix != npix) & (skey != INT_MAX)
    spix_c = spix & (CHUNK_PIX - 1)
    plsc.store_scatter(
        grid, [spix_c >> 9, spix_c & (W - 1)], _f(sp + 1), mask=winner)
    return _
  lax.fori_loop(0, PGROUPS, point_group, None)

  xbase = b * (C * P)
  pltpu.async_copy(x_hbm.at[pl.ds(xbase, P)], pool.at[pl.ds(XC0, P)], sx0)
  pltpu.async_copy(x_hbm.at[pl.ds(xbase + P, P)], pool.at[pl.ds(XC1, P)], sx1)

  def compact(mg, cnt):
    m = _i(grid[mg >> 5, pl.ds((mg & 31) * L, L)])
    msk = m > 0
    mi = msk.astype(jnp.int32)
    pos = jnp.maximum(cnt + plsc.cumsum(mi) - 1, 0)
    entry = ((mg * L + iota) << PBITS) + m - 1
    plsc.store_scatter(
        grid, [LIST_ROW + (pos >> 9), pos & (W - 1)], _f(entry), mask=msk)
    return cnt + jnp.sum(mi)
  cnt = lax.fori_loop(0, MGROUPS, compact, jnp.int32(0))

  pad = cnt + iota
  plsc.store_scatter(
      grid, [LIST_ROW + (pad >> 9), pad & (W - 1)],
      _f(jnp.full((L,), -1, jnp.int32)))
  ngroups = (cnt + (L - 1)) // L

  def zero_stage(i, _):
    grid[i >> 5, pl.ds((i & 31) * L, L)] = fzeros
    return _
  lax.fori_loop(0, 2 * MGROUPS, zero_stage, None)

  def out_dst(c):
    return out_hbm.at[b, c, pl.ds(r0, ROWS_PER_CHUNK)]

  def do_channel(c, xc, srow, sem_x, sem_s):
    pltpu.make_async_copy(
        x_hbm.at[pl.ds(xbase + c * P, P)], pool.at[pl.ds(xc, P)], sem_x).wait()

    @pl.when(c >= 2)
    def _wait_stage():
      pltpu.make_async_copy(
          grid.at[pl.ds(srow, ROWS_PER_CHUNK)], out_dst(c - 2), sem_s).wait()

    def gather_group(g, _):
      e = _i(grid[LIST_ROW + (g >> 5), pl.ds((g & 31) * L, L)])
      live = e >= 0
      pv = e & ((1 << PBITS) - 1)
      li = (e >> PBITS) & (CHUNK_PIX - 1)
      vals = plsc.load_gather(pool, [xc + pv], mask=live)
      plsc.store_scatter(
          grid, [srow + (li >> 9), li & (W - 1)], vals, mask=live)
      return _
    lax.fori_loop(0, ngroups, gather_group, None)

    @pl.when(c < C - 2)
    def _prefetch():
      pltpu.async_copy(
          x_hbm.at[pl.ds(xbase + (c + 2) * P, P)], pool.at[pl.ds(xc, P)],
          sem_x)

    pltpu.async_copy(grid.at[pl.ds(srow, ROWS_PER_CHUNK)], out_dst(c), sem_s)

  def chan_pair(cc, _):
    do_channel(cc * 2, XC0, STG0_ROW, sx0, ss0)
    do_channel(cc * 2 + 1, XC1, STG1_ROW, sx1, ss1)
    return _
  lax.fori_loop(0, C // 2, chan_pair, None)

  pltpu.make_async_copy(
      grid.at[pl.ds(STG0_ROW, ROWS_PER_CHUNK)], out_dst(C - 2), ss0).wait()
  pltpu.make_async_copy(
      grid.at[pl.ds(STG1_ROW, ROWS_PER_CHUNK)], out_dst(C - 1), ss1).wait()


_mesh = plsc.VectorSubcoreMesh(
    core_axis_name="c", subcore_axis_name="s", num_cores=NC, num_subcores=NS)

_sc_scatter = pl.kernel(
    _body,
    out_type=jax.ShapeDtypeStruct((B, C, H, W), jnp.float32),
    mesh=_mesh,
    scratch_types=[
        pltpu.VMEM((POOL,), jnp.float32),
        pltpu.VMEM((GRID_ROWS, W), jnp.float32),
        pltpu.SemaphoreType.DMA,
        pltpu.SemaphoreType.DMA,
        pltpu.SemaphoreType.DMA,
        pltpu.SemaphoreType.DMA,
    ],
    compiler_params=pltpu.CompilerParams(needs_layout_passes=False),
)


@jax.jit
def kernel(x, inds):
  ind_t = lax.bitcast_convert_type(
      jnp.transpose(inds, (2, 0, 1)).reshape(-1), jnp.float32)
  return _sc_scatter(x.reshape(-1), ind_t)
